# baseline (speedup 1.0000x reference)
import numpy as np
import jax
import jax.numpy as jnp
from jax.experimental import pallas as pl
from jax.experimental.pallas import tpu as pltpu

_NUM = 5
_CLASSES = 20
_ALEN = _CLASSES + 5
_H = 16
_W = 16
_P = _H * _W
_N = _NUM * _P
_THRESH = 0.45
_AW = [1.3221, 3.19275, 5.05587, 9.47112, 11.2364]
_AH = [1.73145, 4.00944, 8.09892, 4.84053, 10.0071]
_BLK = 128
_NBLK = _N // _BLK


def _nms_kernel(x_ref, out_ref, s_ref, diag_ref, supp_ref):
    x = x_ref[0]

    iota_p = jax.lax.broadcasted_iota(jnp.int32, (1, _P), 1)
    i_list = (iota_p % _W).astype(jnp.float32)
    j_list = (iota_p // _W).astype(jnp.float32)
    iota20 = jax.lax.broadcasted_iota(jnp.int32, (_CLASSES, _P), 0)

    objs, bxs, bys, bws, bhs, idxs, probs = [], [], [], [], [], [], []
    for a in range(_NUM):
        base = a * _ALEN
        tx = x[base + 0:base + 1, :]
        ty = x[base + 1:base + 2, :]
        tw = x[base + 2:base + 3, :]
        th = x[base + 3:base + 4, :]
        to = x[base + 4:base + 5, :]
        tc = x[base + 5:base + _ALEN, :]
        bxs.append((tx + i_list) / float(_W))
        bys.append((ty + j_list) / float(_H))
        aw = float(np.float32(_AW[a]) / np.float32(_W))
        ah = float(np.float32(_AH[a]) / np.float32(_H))
        bws.append(jnp.exp(tw) * aw)
        bhs.append(jnp.exp(th) * ah)
        objs.append(to)
        pm = jnp.max(tc, axis=0, keepdims=True)
        eq = tc == pm
        idx = jnp.min(jnp.where(eq, iota20, _CLASSES + 1), axis=0,
                      keepdims=True)
        idxs.append(idx.astype(jnp.float32))
        probs.append(pm)

    obj = jnp.concatenate(objs, axis=1)
    bx = jnp.concatenate(bxs, axis=1)
    by = jnp.concatenate(bys, axis=1)
    bw = jnp.concatenate(bws, axis=1)
    bh = jnp.concatenate(bhs, axis=1)
    idxc = jnp.concatenate(idxs, axis=1)
    prob = jnp.concatenate(probs, axis=1)

    det = jnp.concatenate(
        [obj, bx, by, bw, bh, idxc, prob, jnp.zeros((1, _N), jnp.float32)],
        axis=0)

    sT = jnp.transpose(obj)
    iota_row = jax.lax.broadcasted_iota(jnp.int32, (_BLK, _N), 0)
    iota_col = jax.lax.broadcasted_iota(jnp.int32, (_BLK, _N), 1)
    rank_blocks = []
    for kb in range(_NBLK):
        s_col = sT[kb * _BLK:(kb + 1) * _BLK, :]
        gt = obj > s_col
        tie = (obj == s_col) & (iota_col < (kb * _BLK + iota_row))
        a_blk = jnp.where(gt | tie, 1.0, 0.0)
        rank_blocks.append(jnp.sum(a_blk, axis=1, keepdims=True))
    rankT = jnp.concatenate(rank_blocks, axis=0)

    fcol = iota_col.astype(jnp.float32)
    for kb in range(_NBLK):
        r_blk = rankT[kb * _BLK:(kb + 1) * _BLK, :]
        s_ref[kb * _BLK:(kb + 1) * _BLK, :] = jnp.where(
            fcol == r_blk, 1.0, 0.0)

    sorted_rows = jnp.dot(det, s_ref[:, :],
                          preferred_element_type=jnp.float32)

    ss = sorted_rows[0:1, :]
    sbx = sorted_rows[1:2, :]
    sby = sorted_rows[2:3, :]
    sbw = sorted_rows[3:4, :]
    sbh = sorted_rows[4:5, :]

    lx = sbx - 0.5 * sbw
    rx = sbx + 0.5 * sbw
    uy = sby - 0.5 * sbh
    dy = sby + 0.5 * sbh
    area = sbw * sbh

    pack = jnp.concatenate(
        [lx, rx, uy, dy, area, jnp.zeros((3, _N), jnp.float32)], axis=0)
    packT = jnp.transpose(pack)

    for kb in range(_NBLK):
        row0 = kb * _BLK
        lx_c = packT[row0:row0 + _BLK, 0:1]
        rx_c = packT[row0:row0 + _BLK, 1:2]
        uy_c = packT[row0:row0 + _BLK, 2:3]
        dy_c = packT[row0:row0 + _BLK, 3:4]
        ar_c = packT[row0:row0 + _BLK, 4:5]
        left = jnp.maximum(lx_c, lx)
        right = jnp.minimum(rx_c, rx)
        up = jnp.maximum(uy_c, uy)
        down = jnp.minimum(dy_c, dy)
        iw = jnp.maximum(right - left, 0.0)
        ih = jnp.maximum(down - up, 0.0)
        inter = iw * ih
        union = ar_c + area - inter
        iou = inter / union
        upper = iota_col > (row0 + iota_row)
        sup_blk = jnp.where((iou >= _THRESH) & upper, 1.0, 0.0)
        s_ref[row0:row0 + _BLK, :] = sup_blk
        diag_ref[row0:row0 + _BLK, :] = sup_blk[:, row0:row0 + _BLK]

    pos = jnp.where(ss > 0.0, 1.0, 0.0)
    npos = jnp.sum(pos).astype(jnp.int32)
    nblk_active = (npos + (_BLK - 1)) // _BLK
    supp_ref[:, :] = jnp.zeros((_NBLK, _BLK), jnp.float32)
    lane128 = jax.lax.broadcasted_iota(jnp.int32, (1, _BLK), 1)

    def outer(kb, _):
        base = kb * _BLK
        inner_n = jnp.minimum(npos - base, _BLK)
        supp_blk0 = supp_ref[pl.ds(kb, 1), :]

        def inner(i, sb):
            row = diag_ref[pl.ds(base + i, 1), :]
            row_r = pltpu.roll(row, (_BLK - i) % _BLK, axis=1)
            cur = sb[0:1, 0:1]
            sel = jnp.where(cur < 0.5, jnp.maximum(sb, row_r), sb)
            return pltpu.roll(sel, _BLK - 1, axis=1)

        supp_rot = jax.lax.fori_loop(0, inner_n, inner, supp_blk0)
        supp_blk = pltpu.roll(supp_rot, inner_n, axis=1)
        alive = jnp.where((supp_blk < 0.5) & (lane128 < inner_n), 1.0, 0.0)
        contrib = jnp.dot(alive, s_ref[pl.ds(base, _BLK), :],
                          preferred_element_type=jnp.float32)
        contrib2 = contrib.reshape(_NBLK, _BLK)
        supp_ref[:, :] = jnp.maximum(supp_ref[:, :],
                                     jnp.where(contrib2 > 0.0, 1.0, 0.0))
        return 0

    jax.lax.fori_loop(0, nblk_active, outer, 0)

    supp = supp_ref[:, :].reshape(1, _N)
    final_s = jnp.where(supp >= 0.5, jnp.float32(0.0), ss)
    out_ref[0] = jnp.concatenate(
        [final_s, sorted_rows[1:7, :], jnp.zeros((1, _N), jnp.float32)],
        axis=0)


def kernel(pred):
    B = pred.shape[0]
    xr = pred.reshape(B, _NUM * _ALEN, _P)
    out = pl.pallas_call(
        _nms_kernel,
        grid=(B,),
        in_specs=[pl.BlockSpec((1, _NUM * _ALEN, _P), lambda b: (b, 0, 0))],
        out_specs=pl.BlockSpec((1, 8, _N), lambda b: (b, 0, 0)),
        out_shape=jax.ShapeDtypeStruct((B, 8, _N), jnp.float32),
        scratch_shapes=[pltpu.VMEM((_N, _N), jnp.float32),
                        pltpu.VMEM((_N, _BLK), jnp.float32),
                        pltpu.VMEM((_NBLK, _BLK), jnp.float32)],
    )(xr)
    return out[:, :7, :].transpose(0, 2, 1)


# speedup vs baseline: 1.6161x; 1.6161x over previous
import numpy as np
import jax
import jax.numpy as jnp
from jax.experimental import pallas as pl
from jax.experimental.pallas import tpu as pltpu

_NUM = 5
_CLASSES = 20
_ALEN = _CLASSES + 5
_H = 16
_W = 16
_P = _H * _W
_N = _NUM * _P
_THRESH = 0.45
_AW = [1.3221, 3.19275, 5.05587, 9.47112, 11.2364]
_AH = [1.73145, 4.00944, 8.09892, 4.84053, 10.0071]
_BLK = 128
_NBLK = _N // _BLK


def _nms_kernel(x_ref, out_ref, s_ref, diag_ref, supp_ref):
    x = x_ref[0]

    iota_p = jax.lax.broadcasted_iota(jnp.int32, (1, _P), 1)
    i_list = (iota_p % _W).astype(jnp.float32)
    j_list = (iota_p // _W).astype(jnp.float32)
    iota20 = jax.lax.broadcasted_iota(jnp.int32, (_CLASSES, _P), 0)

    objs, bxs, bys, bws, bhs, idxs, probs = [], [], [], [], [], [], []
    for a in range(_NUM):
        base = a * _ALEN
        tx = x[base + 0:base + 1, :]
        ty = x[base + 1:base + 2, :]
        tw = x[base + 2:base + 3, :]
        th = x[base + 3:base + 4, :]
        to = x[base + 4:base + 5, :]
        tc = x[base + 5:base + _ALEN, :]
        bxs.append((tx + i_list) / float(_W))
        bys.append((ty + j_list) / float(_H))
        aw = float(np.float32(_AW[a]) / np.float32(_W))
        ah = float(np.float32(_AH[a]) / np.float32(_H))
        bws.append(jnp.exp(tw) * aw)
        bhs.append(jnp.exp(th) * ah)
        objs.append(to)
        pm = jnp.max(tc, axis=0, keepdims=True)
        eq = tc == pm
        idx = jnp.min(jnp.where(eq, iota20, _CLASSES + 1), axis=0,
                      keepdims=True)
        idxs.append(idx.astype(jnp.float32))
        probs.append(pm)

    obj = jnp.concatenate(objs, axis=1)
    bx = jnp.concatenate(bxs, axis=1)
    by = jnp.concatenate(bys, axis=1)
    bw = jnp.concatenate(bws, axis=1)
    bh = jnp.concatenate(bhs, axis=1)
    idxc = jnp.concatenate(idxs, axis=1)
    prob = jnp.concatenate(probs, axis=1)

    det = jnp.concatenate(
        [obj, bx, by, bw, bh, idxc, prob, jnp.zeros((1, _N), jnp.float32)],
        axis=0)

    sT = jnp.transpose(obj)
    iota_row = jax.lax.broadcasted_iota(jnp.int32, (_BLK, _N), 0)
    iota_col = jax.lax.broadcasted_iota(jnp.int32, (_BLK, _N), 1)
    rank_blocks = []
    for kb in range(_NBLK):
        s_col = sT[kb * _BLK:(kb + 1) * _BLK, :]
        gt = obj > s_col
        tie = (obj == s_col) & (iota_col < (kb * _BLK + iota_row))
        a_blk = jnp.where(gt | tie, 1.0, 0.0)
        rank_blocks.append(jnp.sum(a_blk, axis=1, keepdims=True))
    rankT = jnp.concatenate(rank_blocks, axis=0)

    fcol = iota_col.astype(jnp.float32)
    for kb in range(_NBLK):
        r_blk = rankT[kb * _BLK:(kb + 1) * _BLK, :]
        s_ref[kb * _BLK:(kb + 1) * _BLK, :] = jnp.where(
            fcol == r_blk, 1.0, 0.0)

    sorted_rows = jnp.dot(det, s_ref[:, :],
                          preferred_element_type=jnp.float32)

    ss = sorted_rows[0:1, :]
    sbx = sorted_rows[1:2, :]
    sby = sorted_rows[2:3, :]
    sbw = sorted_rows[3:4, :]
    sbh = sorted_rows[4:5, :]

    lx = sbx - 0.5 * sbw
    rx = sbx + 0.5 * sbw
    uy = sby - 0.5 * sbh
    dy = sby + 0.5 * sbh
    area = sbw * sbh

    pos = jnp.where(ss > 0.0, 1.0, 0.0)
    npos = jnp.sum(pos).astype(jnp.int32)

    pack = jnp.concatenate(
        [lx, rx, uy, dy, area, jnp.zeros((3, _N), jnp.float32)], axis=0)
    packT = jnp.transpose(pack)

    for kb in range(_NBLK):
        row0 = kb * _BLK
        lx_c = packT[row0:row0 + _BLK, 0:1]
        rx_c = packT[row0:row0 + _BLK, 1:2]
        uy_c = packT[row0:row0 + _BLK, 2:3]
        dy_c = packT[row0:row0 + _BLK, 3:4]
        ar_c = packT[row0:row0 + _BLK, 4:5]
        left = jnp.maximum(lx_c, lx)
        right = jnp.minimum(rx_c, rx)
        up = jnp.maximum(uy_c, uy)
        down = jnp.minimum(dy_c, dy)
        iw = jnp.maximum(right - left, 0.0)
        ih = jnp.maximum(down - up, 0.0)
        inter = iw * ih
        union = ar_c + area - inter
        iou = inter / union
        upper = iota_col > (row0 + iota_row)
        row_can_suppress = (row0 + iota_row) < npos
        sup_blk = jnp.where((iou >= _THRESH) & upper & row_can_suppress,
                            1.0, 0.0)
        s_ref[row0:row0 + _BLK, :] = sup_blk
        diag_ref[row0:row0 + _BLK, :] = sup_blk[:, row0:row0 + _BLK]

    nblk_active = (npos + (_BLK - 1)) // _BLK
    supp_ref[:, :] = jnp.zeros((_NBLK, _BLK), jnp.float32)

    for kb in range(_NBLK):
        base = kb * _BLK

        @pl.when(kb < nblk_active)
        def _process_block(kb=kb, base=base):
            sb = supp_ref[kb:kb + 1, :]
            for i in range(_BLK):
                row = diag_ref[base + i:base + i + 1, :]
                cur = sb[0:1, i:i + 1]
                sb = jnp.where(cur < 0.5, jnp.maximum(sb, row), sb)
            alive = jnp.where(sb < 0.5, 1.0, 0.0)
            contrib = jnp.dot(alive, s_ref[base:base + _BLK, :],
                              preferred_element_type=jnp.float32)
            contrib2 = contrib.reshape(_NBLK, _BLK)
            supp_ref[:, :] = jnp.maximum(
                supp_ref[:, :], jnp.where(contrib2 > 0.0, 1.0, 0.0))

    supp = supp_ref[:, :].reshape(1, _N)
    final_s = jnp.where(supp >= 0.5, jnp.float32(0.0), ss)
    out_ref[0] = jnp.concatenate(
        [final_s, sorted_rows[1:7, :], jnp.zeros((1, _N), jnp.float32)],
        axis=0)


def kernel(pred):
    B = pred.shape[0]
    xr = pred.reshape(B, _NUM * _ALEN, _P)
    out = pl.pallas_call(
        _nms_kernel,
        grid=(B,),
        in_specs=[pl.BlockSpec((1, _NUM * _ALEN, _P), lambda b: (b, 0, 0))],
        out_specs=pl.BlockSpec((1, 8, _N), lambda b: (b, 0, 0)),
        out_shape=jax.ShapeDtypeStruct((B, 8, _N), jnp.float32),
        scratch_shapes=[pltpu.VMEM((_N, _N), jnp.float32),
                        pltpu.VMEM((_N, _BLK), jnp.float32),
                        pltpu.VMEM((_NBLK, _BLK), jnp.float32)],
    )(xr)
    return out[:, :7, :].transpose(0, 2, 1)
